# baseline (speedup 1.0000x reference)
import functools

import jax
import jax.numpy as jnp
from jax import lax
from jax.experimental import pallas as pl
from jax.experimental.pallas import tpu as pltpu
from jax.experimental.pallas import tpu_sc as plsc

NUM_CORES = 2
NUM_SUBCORES = 16
NUM_WORKERS = NUM_CORES * NUM_SUBCORES
CHUNK = 16
N_HBM = 7


@functools.lru_cache(maxsize=None)
def _build(bs, pos_len, table_len, d_model):
    rows = bs // NUM_WORKERS
    mesh = plsc.VectorSubcoreMesh(core_axis_name="c", subcore_axis_name="s")

    @functools.partial(
        pl.kernel,
        mesh=mesh,
        out_type=jax.ShapeDtypeStruct((bs, d_model), jnp.float32),
        scratch_types=[
            pltpu.VMEM((pos_len, rows), jnp.int32),
            pltpu.VMEM((rows, d_model), jnp.float32),
            pltpu.VMEM_SHARED((table_len, d_model), jnp.float32),
            pltpu.SemaphoreType.DMA,
            pltpu.SemaphoreType.DMA,
        ],
    )
    def run(x_hbm, post_hbm, tab_hbm, out_hbm, pos_v, acc_v, tab_sh, sem_h, sem_s):
        wid = lax.axis_index("s") * NUM_CORES + lax.axis_index("c")
        base = wid * rows

        sid = lax.axis_index("s")
        shard = table_len // NUM_SUBCORES
        srem = table_len - shard * NUM_SUBCORES
        pltpu.sync_copy(tab_hbm.at[pl.ds(sid * shard, shard)],
                        tab_sh.at[pl.ds(sid * shard, shard)])
        if srem:
            @pl.when(sid == NUM_SUBCORES - 1)
            def _():
                pltpu.sync_copy(
                    tab_hbm.at[pl.ds(shard * NUM_SUBCORES, srem)],
                    tab_sh.at[pl.ds(shard * NUM_SUBCORES, srem)])

        pltpu.sync_copy(post_hbm.at[:, pl.ds(base, rows)], pos_v)
        pltpu.sync_copy(x_hbm.at[pl.ds(base, rows), :], acc_v)
        plsc.subcore_barrier()

        def fire(k, j):
            if j % CHUNK >= N_HBM:
                pltpu.async_copy(tab_sh.at[pos_v.at[k]], acc_v, sem_s,
                                 add=True)
            else:
                pltpu.async_copy(tab_hbm.at[pos_v.at[k]], acc_v, sem_h,
                                 add=True)

        def drain_one(j):
            if j % CHUNK >= N_HBM:
                pltpu.make_async_copy(
                    tab_sh.at[pos_v.at[0]], acc_v, sem_s).wait()
            else:
                pltpu.make_async_copy(
                    tab_hbm.at[pos_v.at[0]], acc_v, sem_h).wait()

        nfull = pos_len // CHUNK
        rem = pos_len - nfull * CHUNK

        def chunk_body(c, carry):
            k0 = c * CHUNK
            for j in range(CHUNK):
                fire(k0 + j, j)

            @pl.when(c > 0)
            def _():
                for j in range(CHUNK):
                    drain_one(j)

            return carry

        lax.fori_loop(0, nfull, chunk_body, 0)
        for j in range(rem):
            fire(nfull * CHUNK + j, j)
        for j in range(CHUNK):
            drain_one(j)
        for j in range(rem):
            drain_one(j)
        pltpu.sync_copy(acc_v, out_hbm.at[pl.ds(base, rows), :])

    return run


def kernel(x, positions, pe):
    table = pe[0]
    table_len, d_model = table.shape
    bs, pos_len = positions.shape
    pos_t = ((positions.astype(jnp.int32) + table_len) % table_len).T
    return _build(bs, pos_len, table_len, d_model)(x, pos_t, table)


# speedup vs baseline: 1.0041x; 1.0041x over previous
import functools

import jax
import jax.numpy as jnp
from jax import lax
from jax.experimental import pallas as pl
from jax.experimental.pallas import tpu as pltpu
from jax.experimental.pallas import tpu_sc as plsc

NUM_CORES = 2
NUM_SUBCORES = 16
NUM_WORKERS = NUM_CORES * NUM_SUBCORES
CHUNK = 20
N_HBM = 9


@functools.lru_cache(maxsize=None)
def _build(bs, pos_len, table_len, d_model):
    rows = bs // NUM_WORKERS
    mesh = plsc.VectorSubcoreMesh(core_axis_name="c", subcore_axis_name="s")

    @functools.partial(
        pl.kernel,
        mesh=mesh,
        out_type=jax.ShapeDtypeStruct((bs, d_model), jnp.float32),
        scratch_types=[
            pltpu.VMEM((pos_len, rows), jnp.int32),
            pltpu.VMEM((rows, d_model), jnp.float32),
            pltpu.VMEM_SHARED((table_len, d_model), jnp.float32),
            pltpu.SemaphoreType.DMA,
            pltpu.SemaphoreType.DMA,
        ],
    )
    def run(x_hbm, post_hbm, tab_hbm, out_hbm, pos_v, acc_v, tab_sh, sem_h, sem_s):
        wid = lax.axis_index("s") * NUM_CORES + lax.axis_index("c")
        base = wid * rows

        sid = lax.axis_index("s")
        shard = table_len // NUM_SUBCORES
        srem = table_len - shard * NUM_SUBCORES
        pltpu.sync_copy(tab_hbm.at[pl.ds(sid * shard, shard)],
                        tab_sh.at[pl.ds(sid * shard, shard)])
        if srem:
            @pl.when(sid == NUM_SUBCORES - 1)
            def _():
                pltpu.sync_copy(
                    tab_hbm.at[pl.ds(shard * NUM_SUBCORES, srem)],
                    tab_sh.at[pl.ds(shard * NUM_SUBCORES, srem)])

        pltpu.sync_copy(post_hbm.at[:, pl.ds(base, rows)], pos_v)
        pltpu.sync_copy(x_hbm.at[pl.ds(base, rows), :], acc_v)
        plsc.subcore_barrier()

        def fire(k, j):
            if j % CHUNK >= N_HBM:
                pltpu.async_copy(tab_sh.at[pos_v.at[k]], acc_v, sem_s,
                                 add=True)
            else:
                pltpu.async_copy(tab_hbm.at[pos_v.at[k]], acc_v, sem_h,
                                 add=True)

        def drain_one(j):
            if j % CHUNK >= N_HBM:
                pltpu.make_async_copy(
                    tab_sh.at[pos_v.at[0]], acc_v, sem_s).wait()
            else:
                pltpu.make_async_copy(
                    tab_hbm.at[pos_v.at[0]], acc_v, sem_h).wait()

        nfull = pos_len // CHUNK
        rem = pos_len - nfull * CHUNK

        def chunk_body(c, carry):
            k0 = c * CHUNK
            for j in range(CHUNK):
                fire(k0 + j, j)

            @pl.when(c > 0)
            def _():
                for j in range(CHUNK):
                    drain_one(j)

            return carry

        lax.fori_loop(0, nfull, chunk_body, 0)
        for j in range(rem):
            fire(nfull * CHUNK + j, j)
        for j in range(CHUNK):
            drain_one(j)
        for j in range(rem):
            drain_one(j)
        pltpu.sync_copy(acc_v, out_hbm.at[pl.ds(base, rows), :])

    return run


def kernel(x, positions, pe):
    table = pe[0]
    table_len, d_model = table.shape
    bs, pos_len = positions.shape
    pos_t = ((positions.astype(jnp.int32) + table_len) % table_len).T
    return _build(bs, pos_len, table_len, d_model)(x, pos_t, table)
